# baseline (speedup 1.0000x reference)
import jax
import jax.numpy as jnp
import numpy as np
from jax.experimental import pallas as pl

_N = 2048
_F = 16
_BLK = 256
_C = 128


def _gcn_block(x_ref, e_ref, dm_ref, w1x_ref, wtile_ref, s_ref, w2_ref, o_ref):
    i = pl.program_id(0)
    x = x_ref[...]
    e = e_ref[...]
    dm = dm_ref[...]
    xw = jnp.dot(x, w1x_ref[...], preferred_element_type=jnp.float32)
    agg = jnp.dot(e, xw, preferred_element_type=jnp.float32)
    shuf = s_ref[...]
    wtile = wtile_ref[...]
    bc = jnp.zeros((_BLK, _F), jnp.float32)
    for u in range(_N // _C):
        e2u = jnp.dot(e[:, u * _C:(u + 1) * _C], shuf,
                      preferred_element_type=jnp.float32)
        su = e2u * dm[:, 2 * u * _C:2 * (u + 1) * _C]
        bc = bc + jnp.dot(su, wtile, preferred_element_type=jnp.float32)
    xi = x_ref[pl.ds(i * _BLK, _BLK), :]
    o_ref[...] = (
        jnp.dot(xi, w2_ref[...], preferred_element_type=jnp.float32) + agg + bc
    )


def kernel(x, edges, distance_matrix, w1, w2):
    w1x = w1[:, :_F].T
    wtile = jnp.tile(w1[:, _F:].T, (_C, 1))
    sm = np.zeros((_C, 2 * _C), np.float32)
    sm[np.arange(_C), 2 * np.arange(_C)] = 1.0
    sm[np.arange(_C), 2 * np.arange(_C) + 1] = 1.0
    shuf = jnp.asarray(sm)
    w2t = w2.T
    dm_r = distance_matrix.reshape(_N, 2 * _N)

    grid = (_N // _BLK,)
    return pl.pallas_call(
        _gcn_block,
        grid=grid,
        in_specs=[
            pl.BlockSpec((_N, _F), lambda i: (0, 0)),
            pl.BlockSpec((_BLK, _N), lambda i: (i, 0)),
            pl.BlockSpec((_BLK, 2 * _N), lambda i: (i, 0)),
            pl.BlockSpec((_F, _F), lambda i: (0, 0)),
            pl.BlockSpec((2 * _C, _F), lambda i: (0, 0)),
            pl.BlockSpec((_C, 2 * _C), lambda i: (0, 0)),
            pl.BlockSpec((_F, _F), lambda i: (0, 0)),
        ],
        out_specs=pl.BlockSpec((_BLK, _F), lambda i: (i, 0)),
        out_shape=jax.ShapeDtypeStruct((_N, _F), jnp.float32),
    )(x, edges, dm_r, w1x, wtile, shuf, w2t)


# speedup vs baseline: 4.6279x; 4.6279x over previous
import jax
import jax.numpy as jnp
from jax.experimental import pallas as pl

_N = 2048
_F = 16
_BLK = 256


def _gcn_block(x_ref, e_ref, dm_ref, w1x_ref, w1d_ref, w2_ref, o_ref):
    i = pl.program_id(0)
    x = x_ref[...]
    e = e_ref[...]
    xw = jnp.dot(x, w1x_ref[...], preferred_element_type=jnp.float32)
    agg = jnp.dot(e, xw, preferred_element_type=jnp.float32)
    d0 = dm_ref[:, 0, :]
    d1 = dm_ref[:, 1, :]
    b0 = jnp.sum(e * d0, axis=1, keepdims=True)
    b1 = jnp.sum(e * d1, axis=1, keepdims=True)
    w1d = w1d_ref[...]
    bc = b0 * w1d[0, :][None, :] + b1 * w1d[1, :][None, :]
    xi = x_ref[pl.ds(i * _BLK, _BLK), :]
    o_ref[...] = (
        jnp.dot(xi, w2_ref[...], preferred_element_type=jnp.float32) + agg + bc
    )


def kernel(x, edges, distance_matrix, w1, w2):
    w1x = w1[:, :_F].T
    w1d = jnp.zeros((8, _F), jnp.float32).at[:2].set(w1[:, _F:].T)
    w2t = w2.T
    dmt = jnp.transpose(distance_matrix, (0, 2, 1))

    grid = (_N // _BLK,)
    return pl.pallas_call(
        _gcn_block,
        grid=grid,
        in_specs=[
            pl.BlockSpec((_N, _F), lambda i: (0, 0)),
            pl.BlockSpec((_BLK, _N), lambda i: (i, 0)),
            pl.BlockSpec((_BLK, 2, _N), lambda i: (i, 0, 0)),
            pl.BlockSpec((_F, _F), lambda i: (0, 0)),
            pl.BlockSpec((8, _F), lambda i: (0, 0)),
            pl.BlockSpec((_F, _F), lambda i: (0, 0)),
        ],
        out_specs=pl.BlockSpec((_BLK, _F), lambda i: (i, 0)),
        out_shape=jax.ShapeDtypeStruct((_N, _F), jnp.float32),
    )(x, edges, dmt, w1x, w1d, w2t)


# speedup vs baseline: 4.6835x; 1.0120x over previous
import jax
import jax.numpy as jnp
from jax.experimental import pallas as pl

_N = 2048
_F = 16
_BLK = 512


def _gcn_block(x_ref, e_ref, dm_ref, w1x_ref, w1d_ref, w2_ref, o_ref):
    i = pl.program_id(0)
    x = x_ref[...]
    e = e_ref[...]
    xw = jnp.dot(x, w1x_ref[...], preferred_element_type=jnp.float32)
    agg = jnp.dot(e, xw, preferred_element_type=jnp.float32)
    d0 = dm_ref[:, 0, :]
    d1 = dm_ref[:, 1, :]
    b0 = jnp.sum(e * d0, axis=1, keepdims=True)
    b1 = jnp.sum(e * d1, axis=1, keepdims=True)
    w1d = w1d_ref[...]
    bc = b0 * w1d[0, :][None, :] + b1 * w1d[1, :][None, :]
    xi = x_ref[pl.ds(i * _BLK, _BLK), :]
    o_ref[...] = (
        jnp.dot(xi, w2_ref[...], preferred_element_type=jnp.float32) + agg + bc
    )


def kernel(x, edges, distance_matrix, w1, w2):
    w1x = w1[:, :_F].T
    w1d = jnp.zeros((8, _F), jnp.float32).at[:2].set(w1[:, _F:].T)
    w2t = w2.T
    dmt = jnp.transpose(distance_matrix, (0, 2, 1))

    grid = (_N // _BLK,)
    return pl.pallas_call(
        _gcn_block,
        grid=grid,
        in_specs=[
            pl.BlockSpec((_N, _F), lambda i: (0, 0)),
            pl.BlockSpec((_BLK, _N), lambda i: (i, 0)),
            pl.BlockSpec((_BLK, 2, _N), lambda i: (i, 0, 0)),
            pl.BlockSpec((_F, _F), lambda i: (0, 0)),
            pl.BlockSpec((8, _F), lambda i: (0, 0)),
            pl.BlockSpec((_F, _F), lambda i: (0, 0)),
        ],
        out_specs=pl.BlockSpec((_BLK, _F), lambda i: (i, 0)),
        out_shape=jax.ShapeDtypeStruct((_N, _F), jnp.float32),
    )(x, edges, dmt, w1x, w1d, w2t)
